# Initial kernel scaffold; baseline (speedup 1.0000x reference)
import jax
import jax.numpy as jnp
from jax.experimental import pallas as pl


def kernel(x, table):
    raise NotImplementedError("write your pallas kernel here")


# speedup vs baseline: 1.5094x; 1.5094x over previous
import jax
import jax.numpy as jnp
from jax import lax
from jax.experimental import pallas as pl
from jax.experimental.pallas import tpu as pltpu


_BLOCK_TOKENS = 512


def _gather_kernel_body(tokens_per_block):
    def body(ids_ref, table_hbm, out_ref, sem):
        base = pl.program_id(0) * tokens_per_block

        def issue(i, carry):
            tok = ids_ref[base + i]
            pltpu.make_async_copy(table_hbm.at[pl.ds(tok, 1), :],
                                  out_ref.at[pl.ds(i, 1), :],
                                  sem).start()
            return carry

        lax.fori_loop(0, tokens_per_block, issue, 0)

        pltpu.make_async_copy(table_hbm.at[pl.ds(0, tokens_per_block), :],
                              out_ref.at[pl.ds(0, tokens_per_block), :],
                              sem).wait()
    return body


def kernel(x, table):
    b, s = x.shape
    v, d = table.shape
    n = b * s
    dtype = table.dtype
    itemsize = jnp.dtype(dtype).itemsize

    t = min(_BLOCK_TOKENS, n)
    flat_ids = x.reshape(n).astype(jnp.int32)

    cost = pl.CostEstimate(
        flops=0, transcendentals=0,
        bytes_accessed=2 * n * d * itemsize + n * 4)

    out_flat = pl.pallas_call(
        _gather_kernel_body(t),
        out_shape=jax.ShapeDtypeStruct((n, d), dtype),
        grid_spec=pltpu.PrefetchScalarGridSpec(
            num_scalar_prefetch=1,
            grid=(n // t,),
            in_specs=[pl.BlockSpec(memory_space=pl.ANY)],
            out_specs=pl.BlockSpec((t, d), lambda i, ids: (i, 0)),
            scratch_shapes=[pltpu.SemaphoreType.DMA],
        ),
        compiler_params=pltpu.CompilerParams(
            dimension_semantics=("parallel",),
            disable_bounds_checks=True),
        cost_estimate=cost,
    )(flat_ids, table)
    return out_flat.reshape(b, s, d)


# speedup vs baseline: 2.0319x; 1.3461x over previous
import jax
import jax.numpy as jnp
from jax import lax
from jax.experimental import pallas as pl
from jax.experimental.pallas import tpu as pltpu


_BLOCK_TOKENS = 512
_ISSUE_UNROLL = 8


def _gather_kernel_body(tokens_per_block, unroll):
    def body(ids_ref, table_hbm, out_ref, sem):
        base = pl.program_id(0) * tokens_per_block

        def issue_chunk(c, carry):
            row = c * unroll
            toks = [ids_ref[base + row + u] for u in range(unroll)]
            for u in range(unroll):
                pltpu.make_async_copy(table_hbm.at[pl.ds(toks[u], 1), :],
                                      out_ref.at[pl.ds(row + u, 1), :],
                                      sem).start()
            return carry

        lax.fori_loop(0, tokens_per_block // unroll, issue_chunk, 0)

        pltpu.make_async_copy(table_hbm.at[pl.ds(0, tokens_per_block), :],
                              out_ref.at[pl.ds(0, tokens_per_block), :],
                              sem).wait()
    return body


def kernel(x, table):
    b, s = x.shape
    v, d = table.shape
    n = b * s
    dtype = table.dtype
    itemsize = jnp.dtype(dtype).itemsize

    t = min(_BLOCK_TOKENS, n)
    flat_ids = x.reshape(n).astype(jnp.int32)

    cost = pl.CostEstimate(
        flops=0, transcendentals=0,
        bytes_accessed=2 * n * d * itemsize + n * 4)

    out_flat = pl.pallas_call(
        _gather_kernel_body(t, _ISSUE_UNROLL),
        out_shape=jax.ShapeDtypeStruct((n, d), dtype),
        grid_spec=pltpu.PrefetchScalarGridSpec(
            num_scalar_prefetch=1,
            grid=(n // t,),
            in_specs=[pl.BlockSpec(memory_space=pl.ANY)],
            out_specs=pl.BlockSpec((t, d), lambda i, ids: (i, 0)),
            scratch_shapes=[pltpu.SemaphoreType.DMA],
        ),
        compiler_params=pltpu.CompilerParams(
            dimension_semantics=("parallel",),
            disable_bounds_checks=True),
        cost_estimate=cost,
    )(flat_ids, table)
    return out_flat.reshape(b, s, d)


# speedup vs baseline: 2.1101x; 1.0385x over previous
import jax
import jax.numpy as jnp
from jax import lax
from jax.experimental import pallas as pl
from jax.experimental.pallas import tpu as pltpu


_BLOCK_TOKENS = 512
_ISSUE_UNROLL = 16


def _gather_kernel_body(tokens_per_block, unroll):
    def body(ids_ref, table_hbm, out_ref, sem):
        base = pl.program_id(0) * tokens_per_block

        def issue_chunk(c, carry):
            row = c * unroll
            toks = [ids_ref[base + row + u] for u in range(unroll)]
            for u in range(unroll):
                pltpu.make_async_copy(table_hbm.at[pl.ds(toks[u], 1), :],
                                      out_ref.at[pl.ds(row + u, 1), :],
                                      sem).start()
            return carry

        lax.fori_loop(0, tokens_per_block // unroll, issue_chunk, 0)

        pltpu.make_async_copy(table_hbm.at[pl.ds(0, tokens_per_block), :],
                              out_ref.at[pl.ds(0, tokens_per_block), :],
                              sem).wait()
    return body


def kernel(x, table):
    b, s = x.shape
    v, d = table.shape
    n = b * s
    dtype = table.dtype
    itemsize = jnp.dtype(dtype).itemsize

    t = min(_BLOCK_TOKENS, n)
    flat_ids = x.reshape(n).astype(jnp.int32)

    cost = pl.CostEstimate(
        flops=0, transcendentals=0,
        bytes_accessed=2 * n * d * itemsize + n * 4)

    out_flat = pl.pallas_call(
        _gather_kernel_body(t, _ISSUE_UNROLL),
        out_shape=jax.ShapeDtypeStruct((n, d), dtype),
        grid_spec=pltpu.PrefetchScalarGridSpec(
            num_scalar_prefetch=1,
            grid=(n // t,),
            in_specs=[pl.BlockSpec(memory_space=pl.ANY)],
            out_specs=pl.BlockSpec((t, d), lambda i, ids: (i, 0)),
            scratch_shapes=[pltpu.SemaphoreType.DMA],
        ),
        compiler_params=pltpu.CompilerParams(
            dimension_semantics=("parallel",),
            disable_bounds_checks=True),
        cost_estimate=cost,
    )(flat_ids, table)
    return out_flat.reshape(b, s, d)


# speedup vs baseline: 2.2765x; 1.0788x over previous
import jax
import jax.numpy as jnp
from jax import lax
from jax.experimental import pallas as pl
from jax.experimental.pallas import tpu as pltpu


_BLOCK_TOKENS = 1024
_ISSUE_UNROLL = 16


def _gather_kernel_body(tokens_per_block, unroll):
    def body(ids_ref, table_hbm, out_ref, sem):
        base = pl.program_id(0) * tokens_per_block

        def issue_chunk(c, carry):
            row = c * unroll
            toks = [ids_ref[base + row + u] for u in range(unroll)]
            for u in range(unroll):
                pltpu.make_async_copy(table_hbm.at[pl.ds(toks[u], 1), :],
                                      out_ref.at[pl.ds(row + u, 1), :],
                                      sem).start()
            return carry

        lax.fori_loop(0, tokens_per_block // unroll, issue_chunk, 0)

        pltpu.make_async_copy(table_hbm.at[pl.ds(0, tokens_per_block), :],
                              out_ref.at[pl.ds(0, tokens_per_block), :],
                              sem).wait()
    return body


def kernel(x, table):
    b, s = x.shape
    v, d = table.shape
    n = b * s
    dtype = table.dtype
    itemsize = jnp.dtype(dtype).itemsize

    t = min(_BLOCK_TOKENS, n)
    flat_ids = x.reshape(n).astype(jnp.int32)

    cost = pl.CostEstimate(
        flops=0, transcendentals=0,
        bytes_accessed=2 * n * d * itemsize + n * 4)

    out_flat = pl.pallas_call(
        _gather_kernel_body(t, _ISSUE_UNROLL),
        out_shape=jax.ShapeDtypeStruct((n, d), dtype),
        grid_spec=pltpu.PrefetchScalarGridSpec(
            num_scalar_prefetch=1,
            grid=(n // t,),
            in_specs=[pl.BlockSpec(memory_space=pl.ANY)],
            out_specs=pl.BlockSpec((t, d), lambda i, ids: (i, 0)),
            scratch_shapes=[pltpu.SemaphoreType.DMA],
        ),
        compiler_params=pltpu.CompilerParams(
            dimension_semantics=("parallel",),
            disable_bounds_checks=True),
        cost_estimate=cost,
    )(flat_ids, table)
    return out_flat.reshape(b, s, d)


# speedup vs baseline: 2.3007x; 1.0107x over previous
import jax
import jax.numpy as jnp
from jax import lax
from jax.experimental import pallas as pl
from jax.experimental.pallas import tpu as pltpu


_BLOCK_TOKENS = 2048
_ISSUE_UNROLL = 16


def _gather_kernel_body(tokens_per_block, unroll):
    def body(ids_ref, table_hbm, out_ref, sem):
        base = pl.program_id(0) * tokens_per_block

        def issue_chunk(c, carry):
            row = c * unroll
            toks = [ids_ref[base + row + u] for u in range(unroll)]
            for u in range(unroll):
                pltpu.make_async_copy(table_hbm.at[pl.ds(toks[u], 1), :],
                                      out_ref.at[pl.ds(row + u, 1), :],
                                      sem).start()
            return carry

        lax.fori_loop(0, tokens_per_block // unroll, issue_chunk, 0)

        pltpu.make_async_copy(table_hbm.at[pl.ds(0, tokens_per_block), :],
                              out_ref.at[pl.ds(0, tokens_per_block), :],
                              sem).wait()
    return body


def kernel(x, table):
    b, s = x.shape
    v, d = table.shape
    n = b * s
    dtype = table.dtype
    itemsize = jnp.dtype(dtype).itemsize

    t = min(_BLOCK_TOKENS, n)
    flat_ids = x.reshape(n).astype(jnp.int32)

    cost = pl.CostEstimate(
        flops=0, transcendentals=0,
        bytes_accessed=2 * n * d * itemsize + n * 4)

    out_flat = pl.pallas_call(
        _gather_kernel_body(t, _ISSUE_UNROLL),
        out_shape=jax.ShapeDtypeStruct((n, d), dtype),
        grid_spec=pltpu.PrefetchScalarGridSpec(
            num_scalar_prefetch=1,
            grid=(n // t,),
            in_specs=[pl.BlockSpec(memory_space=pl.ANY)],
            out_specs=pl.BlockSpec((t, d), lambda i, ids: (i, 0)),
            scratch_shapes=[pltpu.SemaphoreType.DMA],
        ),
        compiler_params=pltpu.CompilerParams(
            dimension_semantics=("parallel",),
            disable_bounds_checks=True),
        cost_estimate=cost,
    )(flat_ids, table)
    return out_flat.reshape(b, s, d)


# speedup vs baseline: 2.3011x; 1.0002x over previous
import jax
import jax.numpy as jnp
from jax import lax
from jax.experimental import pallas as pl
from jax.experimental.pallas import tpu as pltpu


_BLOCK_TOKENS = 2048
_ISSUE_UNROLL = 16


def _gather_kernel_body(tokens_per_block, unroll):
    def body(ids_ref, table_hbm, out_ref, sem):
        base = pl.program_id(0) * tokens_per_block

        def issue_chunk(c, carry):
            row = c * unroll
            toks = [ids_ref[base + row + u] for u in range(unroll)]
            for u in range(unroll):
                pltpu.make_async_copy(table_hbm.at[pl.ds(toks[u], 1), :],
                                      out_ref.at[pl.ds(row + u, 1), :],
                                      sem).start(priority=u % 2)
            return carry

        lax.fori_loop(0, tokens_per_block // unroll, issue_chunk, 0)

        pltpu.make_async_copy(table_hbm.at[pl.ds(0, tokens_per_block), :],
                              out_ref.at[pl.ds(0, tokens_per_block), :],
                              sem).wait()
    return body


def kernel(x, table):
    b, s = x.shape
    v, d = table.shape
    n = b * s
    dtype = table.dtype
    itemsize = jnp.dtype(dtype).itemsize

    t = min(_BLOCK_TOKENS, n)
    flat_ids = x.reshape(n).astype(jnp.int32)

    cost = pl.CostEstimate(
        flops=0, transcendentals=0,
        bytes_accessed=2 * n * d * itemsize + n * 4)

    out_flat = pl.pallas_call(
        _gather_kernel_body(t, _ISSUE_UNROLL),
        out_shape=jax.ShapeDtypeStruct((n, d), dtype),
        grid_spec=pltpu.PrefetchScalarGridSpec(
            num_scalar_prefetch=1,
            grid=(n // t,),
            in_specs=[pl.BlockSpec(memory_space=pl.ANY)],
            out_specs=pl.BlockSpec((t, d), lambda i, ids: (i, 0)),
            scratch_shapes=[pltpu.SemaphoreType.DMA],
        ),
        compiler_params=pltpu.CompilerParams(
            dimension_semantics=("parallel",),
            disable_bounds_checks=True),
        cost_estimate=cost,
    )(flat_ids, table)
    return out_flat.reshape(b, s, d)


# speedup vs baseline: 2.3537x; 1.0229x over previous
import jax
import jax.numpy as jnp
from jax import lax
from jax.experimental import pallas as pl
from jax.experimental.pallas import tpu as pltpu


_BLOCK_TOKENS = 2048
_ISSUE_UNROLL = 32


def _gather_kernel_body(tokens_per_block, unroll):
    def body(ids_ref, table_hbm, out_ref, sem):
        base = pl.program_id(0) * tokens_per_block

        def issue_chunk(c, carry):
            row = c * unroll
            toks = [ids_ref[base + row + u] for u in range(unroll)]
            for u in range(unroll):
                pltpu.make_async_copy(table_hbm.at[pl.ds(toks[u], 1), :],
                                      out_ref.at[pl.ds(row + u, 1), :],
                                      sem).start()
            return carry

        lax.fori_loop(0, tokens_per_block // unroll, issue_chunk, 0)

        pltpu.make_async_copy(table_hbm.at[pl.ds(0, tokens_per_block), :],
                              out_ref.at[pl.ds(0, tokens_per_block), :],
                              sem).wait()
    return body


def kernel(x, table):
    b, s = x.shape
    v, d = table.shape
    n = b * s
    dtype = table.dtype
    itemsize = jnp.dtype(dtype).itemsize

    t = min(_BLOCK_TOKENS, n)
    flat_ids = x.reshape(n).astype(jnp.int32)

    cost = pl.CostEstimate(
        flops=0, transcendentals=0,
        bytes_accessed=2 * n * d * itemsize + n * 4)

    out_flat = pl.pallas_call(
        _gather_kernel_body(t, _ISSUE_UNROLL),
        out_shape=jax.ShapeDtypeStruct((n, d), dtype),
        grid_spec=pltpu.PrefetchScalarGridSpec(
            num_scalar_prefetch=1,
            grid=(n // t,),
            in_specs=[pl.BlockSpec(memory_space=pl.ANY)],
            out_specs=pl.BlockSpec((t, d), lambda i, ids: (i, 0)),
            scratch_shapes=[pltpu.SemaphoreType.DMA],
        ),
        compiler_params=pltpu.CompilerParams(
            dimension_semantics=("parallel",),
            disable_bounds_checks=True),
        cost_estimate=cost,
    )(flat_ids, table)
    return out_flat.reshape(b, s, d)
